# baseline (speedup 1.0000x reference)
import jax
import jax.numpy as jnp
from jax.experimental import pallas as pl
from jax.experimental.pallas import tpu as pltpu

EMB_DIM = 128
N_POS = 128 * 370
NEGATIVE_SIZE = 5
N_NEG = N_POS * NEGATIVE_SIZE
BLOCK = 1280
GRID = N_POS // BLOCK


def _loss_kernel(pu_ref, pv_ref, nu_ref, nv_ref, out_ref):
    step = pl.program_id(0)
    ones = jnp.ones((EMB_DIM, 1), jnp.float32)

    pos_score = jax.lax.dot(pu_ref[...] * pv_ref[...], ones,
                            precision=jax.lax.Precision.HIGHEST)
    pos_score = jnp.clip(pos_score, -6.0, 6.0)
    pos_part = jnp.sum(jnp.log1p(jnp.exp(-pos_score)))

    neg_score = jax.lax.dot(nu_ref[...] * nv_ref[...], ones,
                            precision=jax.lax.Precision.HIGHEST)
    neg_score = jnp.clip(neg_score, -6.0, 6.0)
    neg_part = jnp.sum(jnp.log1p(jnp.exp(neg_score)))

    partial = pos_part + neg_part

    @pl.when(step == 0)
    def _init():
        out_ref[0, 0] = partial

    @pl.when(step != 0)
    def _acc():
        out_ref[0, 0] += partial


def kernel(emb_pos_u, emb_pos_v, emb_neg_u, emb_neg_v):
    pos_spec = pl.BlockSpec((BLOCK, EMB_DIM), lambda i: (i, 0))
    neg_spec = pl.BlockSpec((BLOCK * NEGATIVE_SIZE, EMB_DIM), lambda i: (i, 0))

    total = pl.pallas_call(
        _loss_kernel,
        grid=(GRID,),
        in_specs=[pos_spec, pos_spec, neg_spec, neg_spec],
        out_specs=pl.BlockSpec((1, 1), lambda i: (0, 0),
                               memory_space=pltpu.SMEM),
        out_shape=jax.ShapeDtypeStruct((1, 1), jnp.float32),
    )(emb_pos_u, emb_pos_v, emb_neg_u, emb_neg_v)

    return total[0, 0] / jnp.float32(N_POS)


# speedup vs baseline: 2.2221x; 2.2221x over previous
import jax
import jax.numpy as jnp
from jax.experimental import pallas as pl
from jax.experimental.pallas import tpu as pltpu

EMB_DIM = 128
N_POS = 128 * 370
NEGATIVE_SIZE = 5
N_NEG = N_POS * NEGATIVE_SIZE
BLOCK = 1280
GRID = N_POS // BLOCK


def _loss_kernel(pu_ref, pv_ref, nu_ref, nv_ref, out_ref):
    step = pl.program_id(0)

    def body(u, v, sign):
        n = u.shape[0]
        prod = (u * v).reshape(n // 128, 128, EMB_DIM)
        prod_t = jnp.swapaxes(prod, 1, 2)
        score = jnp.sum(prod_t, axis=1)
        score = jnp.clip(score, -6.0, 6.0)
        return jnp.sum(jnp.log1p(jnp.exp(sign * score)))

    pos_part = body(pu_ref[...], pv_ref[...], -1.0)
    neg_part = body(nu_ref[...], nv_ref[...], 1.0)

    partial = pos_part + neg_part

    @pl.when(step == 0)
    def _init():
        out_ref[0, 0] = partial

    @pl.when(step != 0)
    def _acc():
        out_ref[0, 0] += partial


def kernel(emb_pos_u, emb_pos_v, emb_neg_u, emb_neg_v):
    pos_spec = pl.BlockSpec((BLOCK, EMB_DIM), lambda i: (i, 0))
    neg_spec = pl.BlockSpec((BLOCK * NEGATIVE_SIZE, EMB_DIM), lambda i: (i, 0))

    total = pl.pallas_call(
        _loss_kernel,
        grid=(GRID,),
        in_specs=[pos_spec, pos_spec, neg_spec, neg_spec],
        out_specs=pl.BlockSpec((1, 1), lambda i: (0, 0),
                               memory_space=pltpu.SMEM),
        out_shape=jax.ShapeDtypeStruct((1, 1), jnp.float32),
    )(emb_pos_u, emb_pos_v, emb_neg_u, emb_neg_v)

    return total[0, 0] / jnp.float32(N_POS)
